# baseline (speedup 1.0000x reference)
import functools

import jax
import jax.numpy as jnp
from jax.experimental import pallas as pl
from jax.experimental.pallas import tpu as pltpu

Z_WHAT = 64
DEC = 64
IMG = 416
EMPTY_DEPTH = -1000.0
B, A = 4, 8
SLOT = 128


def _decode_kernel(zw_ref, w1_ref, b1_ref, w2_ref, b2_ref, zd_ref,
                   out_ref, wout_ref):
    h = jnp.dot(zw_ref[...], w1_ref[...], preferred_element_type=jnp.float32)
    h = jnp.maximum(h + b1_ref[...], 0.0)
    o = jnp.dot(h, w2_ref[...], preferred_element_type=jnp.float32)
    o = o + b2_ref[...]
    out_ref[...] = (1.0 / (1.0 + jnp.exp(-o))).astype(jnp.bfloat16)

    d = zd_ref[...]
    m = jnp.max(d, axis=1, keepdims=True)
    e = jnp.exp(d - m)
    denom = jnp.sum(e, axis=1, keepdims=True) + jnp.exp(EMPTY_DEPTH - m)
    wout_ref[...] = e / denom


def _interp_matrix(lin, k, center, scale, weight):
    alpha = (DEC / 2.0) / scale
    beta = (1.0 - center / scale) * (DEC / 2.0) - 0.5
    src = lin * alpha + beta
    t = jnp.abs(k - src)
    return jnp.maximum(weight - weight * t, 0.0).astype(jnp.bfloat16)


def _merge_kernel(zw_ref, w_ref, dec_ref, out_ref, u_ref, rx_ref):
    b = pl.program_id(0)

    @pl.when(b == 0)
    def _zero():
        u_ref[...] = jnp.zeros_like(u_ref)
        rx_ref[...] = jnp.zeros_like(rx_ref)

    lin = jax.lax.broadcasted_iota(jnp.int32, (IMG, DEC), 0).astype(jnp.float32)
    lin = lin * (2.0 / (IMG - 1)) - 1.0
    k = jax.lax.broadcasted_iota(jnp.int32, (IMG, DEC), 1).astype(jnp.float32)

    for a in range(A):
        obj = b * A + a
        cx = zw_ref[obj, 0] * 2.0 - 1.0
        cy = zw_ref[obj, 1] * 2.0 - 1.0
        sx = zw_ref[obj, 2] + 0.05
        sy = zw_ref[obj, 3] + 0.05
        wa = w_ref[b, a]

        rx_ref[:, SLOT * a:SLOT * a + DEC] = _interp_matrix(lin, k, cx, sx, wa)
        Ry = _interp_matrix(lin, k, cy, sy, 1.0)
        for c in range(3):
            u_ref[c, :, SLOT * a:SLOT * a + DEC] = jnp.dot(
                Ry, dec_ref[3 * a + c],
                preferred_element_type=jnp.float32).astype(jnp.bfloat16)

    for c in range(3):
        out_ref[0, c] = jax.lax.dot_general(
            u_ref[c], rx_ref[...], (((1,), (1,)), ((), ())),
            preferred_element_type=jnp.float32)


@functools.partial(jax.jit, static_argnames=("interpret",))
def _run(z_where, z_what, z_depth, W1, b1, W2, b2, interpret=False):
    n = B * A
    decoded, weights = pl.pallas_call(
        _decode_kernel,
        out_shape=(
            jax.ShapeDtypeStruct((n, 3 * DEC * DEC), jnp.bfloat16),
            jax.ShapeDtypeStruct((B, A), jnp.float32),
        ),
        interpret=interpret,
    )(z_what.reshape(n, Z_WHAT), W1, b1.reshape(1, -1), W2, b2.reshape(1, -1),
      z_depth.reshape(B, A))

    dec = decoded.reshape(n * 3, DEC, DEC)

    out = pl.pallas_call(
        _merge_kernel,
        grid=(B,),
        in_specs=[
            pl.BlockSpec(memory_space=pltpu.SMEM),
            pl.BlockSpec(memory_space=pltpu.SMEM),
            pl.BlockSpec((3 * A, DEC, DEC), lambda b: (b, 0, 0)),
        ],
        out_specs=pl.BlockSpec((1, 3, IMG, IMG), lambda b: (b, 0, 0, 0)),
        out_shape=jax.ShapeDtypeStruct((B, 3, IMG, IMG), jnp.float32),
        scratch_shapes=[
            pltpu.VMEM((3, IMG, A * SLOT), jnp.bfloat16),
            pltpu.VMEM((IMG, A * SLOT), jnp.bfloat16),
        ],
        interpret=interpret,
    )(z_where.reshape(n, 4), weights, dec)
    return out


def kernel(z_where, z_present, z_what, z_depth, W1, b1, W2, b2):
    del z_present
    return _run(z_where, z_what, z_depth, W1, b1, W2, b2)


# speedup vs baseline: 1.0145x; 1.0145x over previous
import functools

import jax
import jax.numpy as jnp
from jax.experimental import pallas as pl
from jax.experimental.pallas import tpu as pltpu

Z_WHAT = 64
DEC = 64
IMG = 416
EMPTY_DEPTH = -1000.0
B, A = 4, 8
SLOT = 128


def _decode_kernel(zw_ref, w1_ref, b1_ref, w2_ref, b2_ref, zd_ref,
                   out_ref, wout_ref):
    h = jnp.dot(zw_ref[...], w1_ref[...], preferred_element_type=jnp.float32)
    h = jnp.maximum(h + b1_ref[...], 0.0)
    o = jnp.dot(h, w2_ref[...], preferred_element_type=jnp.float32)
    o = o + b2_ref[...]
    out_ref[...] = 1.0 / (1.0 + jnp.exp(-o))

    d = zd_ref[...]
    m = jnp.max(d, axis=1, keepdims=True)
    e = jnp.exp(d - m)
    denom = jnp.sum(e, axis=1, keepdims=True) + jnp.exp(EMPTY_DEPTH - m)
    wout_ref[...] = e / denom


def _interp_matrix(lin, k, center, scale, weight):
    alpha = (DEC / 2.0) / scale
    beta = (1.0 - center / scale) * (DEC / 2.0) - 0.5
    src = lin * alpha + beta
    t = jnp.abs(k - src)
    return jnp.maximum(weight - weight * t, 0.0)


def _merge_kernel(zw_ref, w_ref, dec_ref, out_ref, u_ref, rx_ref):
    b = pl.program_id(0)

    @pl.when(b == 0)
    def _zero():
        u_ref[...] = jnp.zeros_like(u_ref)
        rx_ref[...] = jnp.zeros_like(rx_ref)

    lin = jax.lax.broadcasted_iota(jnp.int32, (IMG, DEC), 0).astype(jnp.float32)
    lin = lin * (2.0 / (IMG - 1)) - 1.0
    k = jax.lax.broadcasted_iota(jnp.int32, (IMG, DEC), 1).astype(jnp.float32)

    for a in range(A):
        obj = b * A + a
        cx = zw_ref[obj, 0] * 2.0 - 1.0
        cy = zw_ref[obj, 1] * 2.0 - 1.0
        sx = zw_ref[obj, 2] + 0.05
        sy = zw_ref[obj, 3] + 0.05
        wa = w_ref[b, a]

        rx_ref[:, SLOT * a:SLOT * a + DEC] = _interp_matrix(lin, k, cx, sx, wa)
        Ry = _interp_matrix(lin, k, cy, sy, 1.0)
        for c in range(3):
            u_ref[c, :, SLOT * a:SLOT * a + DEC] = jnp.dot(
                Ry, dec_ref[3 * a + c], preferred_element_type=jnp.float32)

    for c in range(3):
        out_ref[0, c] = jax.lax.dot_general(
            u_ref[c], rx_ref[...], (((1,), (1,)), ((), ())),
            preferred_element_type=jnp.float32)


@functools.partial(jax.jit, static_argnames=("interpret",))
def _run(z_where, z_what, z_depth, W1, b1, W2, b2, interpret=False):
    n = B * A
    decoded, weights = pl.pallas_call(
        _decode_kernel,
        out_shape=(
            jax.ShapeDtypeStruct((n, 3 * DEC * DEC), jnp.float32),
            jax.ShapeDtypeStruct((B, A), jnp.float32),
        ),
        interpret=interpret,
    )(z_what.reshape(n, Z_WHAT), W1, b1.reshape(1, -1), W2, b2.reshape(1, -1),
      z_depth.reshape(B, A))

    dec = decoded.reshape(n * 3, DEC, DEC)

    out = pl.pallas_call(
        _merge_kernel,
        grid=(B,),
        in_specs=[
            pl.BlockSpec(memory_space=pltpu.SMEM),
            pl.BlockSpec(memory_space=pltpu.SMEM),
            pl.BlockSpec((3 * A, DEC, DEC), lambda b: (b, 0, 0)),
        ],
        out_specs=pl.BlockSpec((1, 3, IMG, IMG), lambda b: (b, 0, 0, 0)),
        out_shape=jax.ShapeDtypeStruct((B, 3, IMG, IMG), jnp.float32),
        scratch_shapes=[
            pltpu.VMEM((3, IMG, A * SLOT), jnp.float32),
            pltpu.VMEM((IMG, A * SLOT), jnp.float32),
        ],
        interpret=interpret,
    )(z_where.reshape(n, 4), weights, dec)
    return out


def kernel(z_where, z_present, z_what, z_depth, W1, b1, W2, b2):
    del z_present
    return _run(z_where, z_what, z_depth, W1, b1, W2, b2)


# speedup vs baseline: 1.0152x; 1.0007x over previous
import functools

import jax
import jax.numpy as jnp
from jax.experimental import pallas as pl
from jax.experimental.pallas import tpu as pltpu

Z_WHAT = 64
DEC = 64
IMG = 416
EMPTY_DEPTH = -1000.0
B, A = 4, 8
SLOT = 128


def _decode_kernel(zw_ref, w1_ref, b1_ref, w2_ref, b2_ref, zd_ref,
                   out_ref, wout_ref):
    h = jnp.dot(zw_ref[...], w1_ref[...], preferred_element_type=jnp.float32)
    h = jnp.maximum(h + b1_ref[...], 0.0)
    o = jnp.dot(h, w2_ref[...], preferred_element_type=jnp.float32)
    o = o + b2_ref[...]
    out_ref[...] = 1.0 / (1.0 + jnp.exp(-o))

    d = zd_ref[...]
    m = jnp.max(d, axis=1, keepdims=True)
    e = jnp.exp(d - m)
    denom = jnp.sum(e, axis=1, keepdims=True) + jnp.exp(EMPTY_DEPTH - m)
    wout_ref[...] = e / denom


def _interp_matrix(lin, k, center, scale, weight):
    alpha = (DEC / 2.0) / scale
    beta = (1.0 - center / scale) * (DEC / 2.0) - 0.5
    src = lin * alpha + beta
    t = jnp.abs(k - src)
    return jnp.maximum(weight - weight * t, 0.0)


def _merge_kernel(zw_ref, w_ref, dec_ref, out_ref, u_ref, rx_ref):
    b = pl.program_id(0)

    @pl.when(b == 0)
    def _zero():
        u_ref[...] = jnp.zeros_like(u_ref)
        rx_ref[...] = jnp.zeros_like(rx_ref)

    lin = jax.lax.broadcasted_iota(jnp.int32, (IMG, DEC), 0).astype(jnp.float32)
    lin = lin * (2.0 / (IMG - 1)) - 1.0
    k = jax.lax.broadcasted_iota(jnp.int32, (IMG, DEC), 1).astype(jnp.float32)

    for a in range(A):
        obj = b * A + a
        cx = zw_ref[obj, 0] * 2.0 - 1.0
        cy = zw_ref[obj, 1] * 2.0 - 1.0
        sx = zw_ref[obj, 2] + 0.05
        sy = zw_ref[obj, 3] + 0.05
        wa = w_ref[b, a]

        rx_ref[:, SLOT * a:SLOT * a + DEC] = _interp_matrix(lin, k, cx, sx, wa)
        Ry = _interp_matrix(lin, k, cy, sy, 1.0)
        for c in range(3):
            u_ref[c, :, SLOT * a:SLOT * a + DEC] = jnp.dot(
                Ry, dec_ref[3 * a + c], preferred_element_type=jnp.float32)

    for c in range(3):
        out_ref[0, c] = jax.lax.dot_general(
            u_ref[c], rx_ref[...], (((1,), (1,)), ((), ())),
            preferred_element_type=jnp.float32)


@jax.jit
def _run(z_where, z_what, z_depth, W1, b1, W2, b2):
    n = B * A
    decoded, weights = pl.pallas_call(
        _decode_kernel,
        out_shape=(
            jax.ShapeDtypeStruct((n, 3 * DEC * DEC), jnp.float32),
            jax.ShapeDtypeStruct((B, A), jnp.float32),
        ),
    )(z_what.reshape(n, Z_WHAT), W1, b1.reshape(1, -1), W2, b2.reshape(1, -1),
      z_depth.reshape(B, A))

    dec = decoded.reshape(n * 3, DEC, DEC)

    out = pl.pallas_call(
        _merge_kernel,
        grid=(B,),
        in_specs=[
            pl.BlockSpec(memory_space=pltpu.SMEM),
            pl.BlockSpec(memory_space=pltpu.SMEM),
            pl.BlockSpec((3 * A, DEC, DEC), lambda b: (b, 0, 0)),
        ],
        out_specs=pl.BlockSpec((1, 3, IMG, IMG), lambda b: (b, 0, 0, 0)),
        out_shape=jax.ShapeDtypeStruct((B, 3, IMG, IMG), jnp.float32),
        scratch_shapes=[
            pltpu.VMEM((3, IMG, A * SLOT), jnp.float32),
            pltpu.VMEM((IMG, A * SLOT), jnp.float32),
        ],
    )(z_where.reshape(n, 4), weights, dec)
    return out


def kernel(z_where, z_present, z_what, z_depth, W1, b1, W2, b2):
    del z_present
    return _run(z_where, z_what, z_depth, W1, b1, W2, b2)
